# baseline (speedup 1.0000x reference)
import functools

import jax
import jax.numpy as jnp
from jax.experimental import pallas as pl
from jax.experimental.pallas import tpu as pltpu

HIDDEN = 256
EH = 128
TOK = 768
N_ENT = 10000
N_REL = 8
N_MOVIE = 5000


def _rgcn_dense_body(comp_ref, basis_ref, x_ref, rootw_ref, rootb_ref,
                     xr_ref, rt_ref):
    r = pl.program_id(0)
    nr = comp_ref.shape[0]
    sel = (jax.lax.broadcasted_iota(jnp.int32, (1, nr), 1) == r)
    row = jnp.where(sel, 1.0, 0.0) @ comp_ref[...]
    W = jnp.tensordot(row, basis_ref[...], axes=((1,), (0,)))[0]
    x = x_ref[...]
    xr_ref[...] = (x @ W)[None]

    @pl.when(r == 0)
    def _():
        rt_ref[...] = x @ rootw_ref[...] + rootb_ref[...][None] + x


def _rgcn_dense(x, basis, comp, root_w, root_b):
    n = x.shape[0]
    return pl.pallas_call(
        _rgcn_dense_body,
        grid=(N_REL,),
        in_specs=[
            pl.BlockSpec(comp.shape, lambda r: (0, 0)),
            pl.BlockSpec(basis.shape, lambda r: (0, 0, 0)),
            pl.BlockSpec(x.shape, lambda r: (0, 0)),
            pl.BlockSpec(root_w.shape, lambda r: (0, 0)),
            pl.BlockSpec(root_b.shape, lambda r: (0,)),
        ],
        out_specs=[
            pl.BlockSpec((1, n, EH), lambda r: (r, 0, 0)),
            pl.BlockSpec((n, EH), lambda r: (0, 0)),
        ],
        out_shape=[
            jax.ShapeDtypeStruct((N_REL, n, EH), jnp.float32),
            jax.ShapeDtypeStruct((n, EH), jnp.float32),
        ],
    )(comp, basis, x, root_w, root_b)


def _ent_mlp_body(x_ref, w1_ref, b1_ref, w2_ref, b2_ref, w3_ref, b3_ref,
                  out_ref):
    x = x_ref[...]
    h = jnp.maximum(x @ w1_ref[...] + b1_ref[...][None], 0.0)
    h = h @ w2_ref[...] + b2_ref[...][None] + x
    out_ref[...] = h @ w3_ref[...] + b3_ref[...][None]


def _ent_mlp(x, w1, b1, w2, b2, w3, b3):
    n = x.shape[0]
    return pl.pallas_call(
        _ent_mlp_body,
        out_shape=jax.ShapeDtypeStruct((n, w3.shape[1]), jnp.float32),
    )(x, w1, b1, w2, b2, w3, b3)


def _attn_body(t_ref, e_ref, cw_ref, out_ref):
    t = t_ref[0]
    e = e_ref[0]
    a = (t @ cw_ref[...]) @ e.T * (1.0 / HIDDEN)
    a = a - jnp.max(a, axis=1, keepdims=True)
    ex = jnp.exp(a)
    ew = ex / jnp.sum(ex, axis=1, keepdims=True)
    out_ref[...] = (ew @ e + t)[None]


def _attention(t, e, cross_w):
    b, lt, _ = t.shape
    le = e.shape[1]
    return pl.pallas_call(
        _attn_body,
        grid=(b,),
        in_specs=[
            pl.BlockSpec((1, lt, HIDDEN), lambda i: (i, 0, 0)),
            pl.BlockSpec((1, le, HIDDEN), lambda i: (i, 0, 0)),
            pl.BlockSpec(cross_w.shape, lambda i: (0, 0)),
        ],
        out_specs=pl.BlockSpec((1, lt, HIDDEN), lambda i: (i, 0, 0)),
        out_shape=jax.ShapeDtypeStruct((b, lt, HIDDEN), jnp.float32),
    )(t, e, cross_w)


def _gcn_pass(x, ei):
    n = x.shape[0]
    loops = jnp.arange(n, dtype=ei.dtype)
    row = jnp.concatenate([ei[0], loops])
    col = jnp.concatenate([ei[1], loops])
    deg = jax.ops.segment_sum(jnp.ones(row.shape[0], x.dtype), col, n)
    dinv = jnp.where(deg > 0, deg ** -0.5, 0.0)
    norm = dinv[row] * dinv[col]
    return jax.ops.segment_sum(norm[:, None] * x[row], col, n)


def kernel(node_embeds, basis, comp, root_w, root_b, ep1_w1, ep1_b1, ep1_w2,
           ep1_b2, ep2_w, ep2_b, tp1_w1, tp1_b1, tp1_w2, tp1_b2, tp2_w, tp2_b,
           cross_w, token_embeds, entity_ids, edge_index, edge_type,
           edge_index_c, edge_index_t_s, edge_index_i_s, movie_indices):
    x = node_embeds
    n = x.shape[0]

    xr, rt = _rgcn_dense(x, basis, comp, root_w, root_b)
    src, dst = edge_index[0], edge_index[1]
    et = edge_type
    seg = dst * N_REL + et
    cnt = jax.ops.segment_sum(jnp.ones(et.shape[0], jnp.float32), seg,
                              n * N_REL)
    inv = 1.0 / jnp.maximum(cnt, 1.0)
    msg = xr[et, src] * inv[seg][:, None]
    ent = jax.ops.segment_sum(msg, dst, n) + rt

    nf = ent[movie_indices]
    ts1 = _gcn_pass(nf, edge_index_t_s)
    ts2 = _gcn_pass(ts1, edge_index_t_s)
    is1 = _gcn_pass(nf, edge_index_i_s)
    is2 = _gcn_pass(is1, edge_index_i_s)
    mean_t = (ts1 + ts2 + is1 + is2) * 0.25
    c1 = _gcn_pass(ent, edge_index_c)
    c2 = _gcn_pass(c1, edge_index_c)
    c3 = _gcn_pass(c2, edge_index_c)
    ent = (c1 + c2 + c3 + ent) * 0.25
    ent = ent.at[movie_indices].add(mean_t)

    ent = _ent_mlp(ent, ep1_w1, ep1_b1, ep1_w2, ep1_b2, ep2_w, ep2_b)

    b, lt, tok = token_embeds.shape
    tflat = token_embeds.reshape(b * lt, tok)
    t2 = _ent_mlp(tflat, tp1_w1, tp1_b1, tp1_w2, tp1_b2, tp2_w, tp2_b)
    t2 = t2.reshape(b, lt, HIDDEN)
    e = ent[entity_ids]
    return _attention(t2, e, cross_w)


# speedup vs baseline: 5.0135x; 5.0135x over previous
import functools

import jax
import jax.numpy as jnp
from jax import lax
from jax.experimental import pallas as pl
from jax.experimental.pallas import tpu as pltpu
from jax.experimental.pallas import tpu_sc as plsc

HIDDEN = 256
EH = 128
N_REL = 8

NC, NSUB, G = 2, 16, 128
NW = NC * NSUB

NP = 10112
SNP = 5056
SN2 = 2 * SNP
BINS = 100352

_MESH = plsc.VectorSubcoreMesh(core_axis_name="c", subcore_axis_name="s")


def _sc_hist(idx3, ones_g, zeros_b, steps):
    bpt = BINS // NSUB

    def body(idx_h, ones_h, zero_h, out_h, idx_v, ones_v, acc_s):
        cid = lax.axis_index("c")
        sid = lax.axis_index("s")
        wid = cid * NSUB + sid
        pltpu.sync_copy(idx_h.at[pl.ds(wid * steps, steps)], idx_v)
        pltpu.sync_copy(ones_h, ones_v)
        b0 = sid * bpt
        pltpu.sync_copy(zero_h.at[pl.ds(b0, bpt)], acc_s.at[pl.ds(b0, bpt)])
        plsc.subcore_barrier()

        def step(j, carry):
            pltpu.sync_copy(ones_v, acc_s.at[idx_v.at[j]], add=True)
            return carry

        lax.fori_loop(0, steps, step, 0)
        plsc.subcore_barrier()
        pltpu.sync_copy(acc_s.at[pl.ds(b0, bpt)],
                        out_h.at[pl.ds(cid * BINS + b0, bpt)])

    k = pl.kernel(
        body,
        out_type=jax.ShapeDtypeStruct((NC * BINS,), jnp.float32),
        mesh=_MESH,
        scratch_types=[
            pltpu.VMEM((steps, G), jnp.int32),
            pltpu.VMEM((G,), jnp.float32),
            pltpu.VMEM_SHARED((BINS,), jnp.float32),
        ],
    )
    return k(idx3, ones_g, zeros_b)


def _edge_prop(table, src3, dst3, zeros, n_pad, steps, inv=None, seg3=None):
    weighted = inv is not None
    rpt = n_pad // NSUB

    def body(table_h, src_h, dst_h, zero_h, *rest):
        if weighted:
            (inv_h, seg_h, out_h, src_v, dst_v, buf_v, acc_s, sem,
             seg_v, w_v, sem2) = rest
        else:
            out_h, src_v, dst_v, buf_v, acc_s, sem = rest
        cid = lax.axis_index("c")
        sid = lax.axis_index("s")
        wid = cid * NSUB + sid
        pltpu.sync_copy(src_h.at[pl.ds(wid * steps, steps)], src_v)
        pltpu.sync_copy(dst_h.at[pl.ds(wid * steps, steps)], dst_v)
        if weighted:
            pltpu.sync_copy(seg_h.at[pl.ds(wid * steps, steps)], seg_v)
        r0 = sid * rpt
        pltpu.sync_copy(zero_h.at[pl.ds(r0, rpt)], acc_s.at[pl.ds(r0, rpt)])
        plsc.subcore_barrier()

        def step(j, carry):
            pltpu.async_copy(table_h.at[src_v.at[j]], buf_v, sem).wait()
            if weighted:
                pltpu.async_copy(inv_h.at[seg_v.at[j]], w_v, sem2).wait()

                def srow(r, c2):
                    w = w_v[pl.ds(r, 1)][0]
                    for q in range(EH // 16):
                        sl = pl.ds(q * 16, 16)
                        buf_v[r, sl] = buf_v[r, sl] * w
                    return c2

                lax.fori_loop(0, G, srow, 0)
            pltpu.sync_copy(buf_v, acc_s.at[dst_v.at[j]], add=True)
            return carry

        lax.fori_loop(0, steps, step, 0)
        plsc.subcore_barrier()
        pltpu.sync_copy(acc_s.at[pl.ds(r0, rpt)],
                        out_h.at[pl.ds(cid * n_pad + r0, rpt)])

    scratch = [
        pltpu.VMEM((steps, G), jnp.int32),
        pltpu.VMEM((steps, G), jnp.int32),
        pltpu.VMEM((G, EH), jnp.float32),
        pltpu.VMEM_SHARED((n_pad, EH), jnp.float32),
        pltpu.SemaphoreType.DMA,
    ]
    if weighted:
        scratch += [
            pltpu.VMEM((steps, G), jnp.int32),
            pltpu.VMEM((G,), jnp.float32),
            pltpu.SemaphoreType.DMA,
        ]
    k = pl.kernel(
        body,
        out_type=jax.ShapeDtypeStruct((NC * n_pad, EH), jnp.float32),
        mesh=_MESH,
        scratch_types=scratch,
    )
    args = (table, src3, dst3, zeros) + ((inv, seg3) if weighted else ())
    return k(*args)


def _sc_gather(table, idx, r_pad, d):
    rpw = r_pad // NW

    def body(table_h, idx_h, out_h, idx_v, rows_v, sem):
        cid = lax.axis_index("c")
        sid = lax.axis_index("s")
        wid = cid * NSUB + sid
        base = wid * rpw
        pltpu.sync_copy(idx_h.at[pl.ds(base, rpw)], idx_v)
        pltpu.async_copy(table_h.at[idx_v], rows_v, sem).wait()
        pltpu.sync_copy(rows_v, out_h.at[pl.ds(base, rpw)])

    k = pl.kernel(
        body,
        out_type=jax.ShapeDtypeStruct((r_pad, d), jnp.float32),
        mesh=_MESH,
        scratch_types=[
            pltpu.VMEM((rpw,), jnp.int32),
            pltpu.VMEM((rpw, d), jnp.float32),
            pltpu.SemaphoreType.DMA,
        ],
    )
    return k(table, idx)


def _rgcn_dense_body(comp_ref, basis_ref, x_ref, rootw_ref, rootb_ref,
                     xr_ref, rt_ref):
    r = pl.program_id(0)
    nr = comp_ref.shape[0]
    sel = (jax.lax.broadcasted_iota(jnp.int32, (1, nr), 1) == r)
    row = jnp.where(sel, 1.0, 0.0) @ comp_ref[...]
    W = jnp.tensordot(row, basis_ref[...], axes=((1,), (0,)))[0]
    x = x_ref[...]
    xr_ref[...] = (x @ W)[None]

    @pl.when(r == 0)
    def _():
        rt_ref[...] = x @ rootw_ref[...] + rootb_ref[...][None] + x


def _rgcn_dense(x, basis, comp, root_w, root_b):
    n = x.shape[0]
    return pl.pallas_call(
        _rgcn_dense_body,
        grid=(N_REL,),
        in_specs=[
            pl.BlockSpec(comp.shape, lambda r: (0, 0)),
            pl.BlockSpec(basis.shape, lambda r: (0, 0, 0)),
            pl.BlockSpec(x.shape, lambda r: (0, 0)),
            pl.BlockSpec(root_w.shape, lambda r: (0, 0)),
            pl.BlockSpec(root_b.shape, lambda r: (0,)),
        ],
        out_specs=[
            pl.BlockSpec((1, n, EH), lambda r: (r, 0, 0)),
            pl.BlockSpec((n, EH), lambda r: (0, 0)),
        ],
        out_shape=[
            jax.ShapeDtypeStruct((N_REL, n, EH), jnp.float32),
            jax.ShapeDtypeStruct((n, EH), jnp.float32),
        ],
    )(comp, basis, x, root_w, root_b)


def _norms_body(h_ref, out_ref):
    v = h_ref[0] + h_ref[1]
    r, c = v.shape
    pos = (jax.lax.broadcasted_iota(jnp.int32, (r, c), 0) * c
           + jax.lax.broadcasted_iota(jnp.int32, (r, c), 1))
    inv = 1.0 / jnp.maximum(v, 1.0)
    dinv = jax.lax.rsqrt(v + 1.0)
    out_ref[...] = jnp.where(pos < 80000, inv, dinv)


def _tc_norms(hist2):
    return pl.pallas_call(
        _norms_body,
        out_shape=jax.ShapeDtypeStruct(hist2.shape[1:], jnp.float32),
    )(hist2)


def _comb_a_body(p0_ref, p1_ref, rt_ref, dinv_ref, ent_ref, y_ref):
    ent = p0_ref[...] + p1_ref[...] + rt_ref[...]
    ent_ref[...] = ent
    y_ref[...] = dinv_ref[...] * ent


def _comb_b_body(p0_ref, p1_ref, y_ref, dinv_ref, c_ref, y2_ref):
    d = dinv_ref[...]
    c = d * (p0_ref[...] + p1_ref[...] + y_ref[...])
    c_ref[...] = c
    y2_ref[...] = d * c


def _row_comb(body, outs, *arrs):
    n = arrs[0].shape[0]
    blk = n // 4
    specs = [pl.BlockSpec((blk, a.shape[1]), lambda i: (i, 0)) for a in arrs]
    return pl.pallas_call(
        body,
        grid=(4,),
        in_specs=specs,
        out_specs=[pl.BlockSpec((blk, EH), lambda i: (i, 0))] * outs,
        out_shape=[jax.ShapeDtypeStruct((n, EH), jnp.float32)] * outs,
    )(*arrs)


def _comb_c_body(nf_ref, dinv_ref, out_ref):
    out_ref[...] = dinv_ref[...] * nf_ref[...]


def _tc_comb_c(nf, dinv_s):
    return pl.pallas_call(
        _comb_c_body,
        grid=(2,),
        in_specs=[
            pl.BlockSpec((SNP, EH), lambda i: (0, 0)),
            pl.BlockSpec((SNP, 1), lambda i: (i, 0)),
        ],
        out_specs=pl.BlockSpec((SNP, EH), lambda i: (i, 0)),
        out_shape=jax.ShapeDtypeStruct((SN2, EH), jnp.float32),
    )(nf, dinv_s)


def _comb_d_body(c1_ref, c2_ref, c3_ref, e0_ref, out_ref):
    out_ref[...] = 0.25 * (c1_ref[...] + c2_ref[...] + c3_ref[...]
                           + e0_ref[...])


def _comb_e_body(a_ref, b_ref, out_ref):
    out_ref[...] = 0.25 * (a_ref[0:SNP] + a_ref[SNP:SN2]
                           + b_ref[0:SNP] + b_ref[SNP:SN2])


def _tc_comb_e(cs1, cs2):
    return pl.pallas_call(
        _comb_e_body,
        out_shape=jax.ShapeDtypeStruct((SNP, EH), jnp.float32),
    )(cs1, cs2)


def _mlp_body(nx, x_refs_and_w):
    def body(*refs):
        xs = refs[:nx]
        w1_ref, b1_ref, w2_ref, b2_ref, w3_ref, b3_ref, out_ref = refs[nx:]
        x = xs[0][...]
        for xr in xs[1:]:
            x = x + xr[...]
        h = jnp.maximum(x @ w1_ref[...] + b1_ref[...][None], 0.0)
        h = h @ w2_ref[...] + b2_ref[...][None] + x
        out_ref[...] = h @ w3_ref[...] + b3_ref[...][None]
    return body


def _mlp_proj(xs, w1, b1, w2, b2, w3, b3, grid=1):
    n, din = xs[0].shape
    dout = w3.shape[1]
    blk = n // grid
    specs = [pl.BlockSpec((blk, din), lambda i: (i, 0)) for _ in xs]
    specs += [pl.BlockSpec(w.shape, lambda i, r=len(w.shape): (0,) * r)
              for w in (w1, b1, w2, b2, w3, b3)]
    return pl.pallas_call(
        _mlp_body(len(xs), None),
        grid=(grid,),
        in_specs=specs,
        out_specs=pl.BlockSpec((blk, dout), lambda i: (i, 0)),
        out_shape=jax.ShapeDtypeStruct((n, dout), jnp.float32),
    )(*xs, w1, b1, w2, b2, w3, b3)


def _attn_body(t_ref, e_ref, cw_ref, out_ref):
    t = t_ref[0]
    e = e_ref[0]
    a = (t @ cw_ref[...]) @ e.T * (1.0 / HIDDEN)
    a = a - jnp.max(a, axis=1, keepdims=True)
    ex = jnp.exp(a)
    ew = ex / jnp.sum(ex, axis=1, keepdims=True)
    out_ref[...] = (ew @ e + t)[None]


def _attention(t, e, cross_w):
    b, lt, _ = t.shape
    le = e.shape[1]
    return pl.pallas_call(
        _attn_body,
        grid=(b,),
        in_specs=[
            pl.BlockSpec((1, lt, HIDDEN), lambda i: (i, 0, 0)),
            pl.BlockSpec((1, le, HIDDEN), lambda i: (i, 0, 0)),
            pl.BlockSpec(cross_w.shape, lambda i: (0, 0)),
        ],
        out_specs=pl.BlockSpec((1, lt, HIDDEN), lambda i: (i, 0, 0)),
        out_shape=jax.ShapeDtypeStruct((b, lt, HIDDEN), jnp.float32),
    )(t, e, cross_w)


def _pad_edge(v, total, fill):
    return jnp.concatenate(
        [v, jnp.full((total - v.shape[0],), fill, jnp.int32)]
    ).reshape(-1, G)


def _steps(e):
    return -(-e // (NW * G * 8)) * 8


def kernel(node_embeds, basis, comp, root_w, root_b, ep1_w1, ep1_b1, ep1_w2,
           ep1_b2, ep2_w, ep2_b, tp1_w1, tp1_b1, tp1_w2, tp1_b2, tp2_w, tp2_b,
           cross_w, token_embeds, entity_ids, edge_index, edge_type,
           edge_index_c, edge_index_t_s, edge_index_i_s, movie_indices):
    n = node_embeds.shape[0]
    nm = movie_indices.shape[0]
    x = jnp.pad(node_embeds, ((0, NP - n), (0, 0)))

    xr, rt = _rgcn_dense(x, basis, comp, root_w, root_b)
    xr_flat = xr.reshape(N_REL * NP, EH)

    src, dst, et = edge_index[0], edge_index[1], edge_type
    e_kg = src.shape[0]
    fe = et * NP + src
    seg = dst * N_REL + et

    hseg = jnp.concatenate([
        seg,
        80000 + edge_index_c[1],
        90000 + edge_index_t_s[1],
        95000 + edge_index_i_s[1],
    ])
    sh = _steps(hseg.shape[0])
    hidx = _pad_edge(hseg, NW * G * sh, BINS - 1)
    ones_g = jnp.ones((G,), jnp.float32)
    zeros_b = jnp.zeros((BINS,), jnp.float32)
    hist2 = _sc_hist(hidx, ones_g, zeros_b, sh)
    norms = _tc_norms(hist2.reshape(2, BINS // 128, 128)).reshape(-1)
    inv = norms[:80000]
    dinv_c = jnp.pad(norms[80000:90000], (0, NP - n),
                     constant_values=1.0)[:, None]
    dinv_s = jnp.concatenate([
        jnp.pad(norms[90000:95000], (0, SNP - nm), constant_values=1.0),
        jnp.pad(norms[95000:100000], (0, SNP - nm), constant_values=1.0),
    ])[:, None]

    s_kg = _steps(e_kg)
    epad = NW * G * s_kg
    zeros_np = jnp.zeros((NP, EH), jnp.float32)
    pr = _edge_prop(xr_flat, _pad_edge(fe, epad, 0), _pad_edge(dst, epad, n),
                    zeros_np, NP, s_kg, inv=inv,
                    seg3=_pad_edge(seg, epad, 0))
    ent0, y0c = _row_comb(_comb_a_body, 2, pr[:NP], pr[NP:], rt, dinv_c)

    e_c = edge_index_c.shape[1]
    s_c = _steps(e_c)
    cpad = NW * G * s_c
    csrc = _pad_edge(edge_index_c[0], cpad, 0)
    cdst = _pad_edge(edge_index_c[1], cpad, n)
    p = _edge_prop(y0c, csrc, cdst, zeros_np, NP, s_c)
    c1, y1 = _row_comb(_comb_b_body, 2, p[:NP], p[NP:], y0c, dinv_c)
    p = _edge_prop(y1, csrc, cdst, zeros_np, NP, s_c)
    c2, y2 = _row_comb(_comb_b_body, 2, p[:NP], p[NP:], y1, dinv_c)
    p = _edge_prop(y2, csrc, cdst, zeros_np, NP, s_c)
    c3, _ = _row_comb(_comb_b_body, 2, p[:NP], p[NP:], y2, dinv_c)
    ent_q = _row_comb(_comb_d_body, 1, c1, c2, c3, ent0)[0]

    mi_pad = jnp.concatenate(
        [movie_indices, jnp.zeros((5120 - nm,), jnp.int32)])
    nf = _sc_gather(ent0, mi_pad, 5120, EH)[:SNP]
    y0s = _tc_comb_c(nf, dinv_s)
    e_s = edge_index_t_s.shape[1]
    s_s = _steps(2 * e_s)
    spad = NW * G * s_s
    ssrc = _pad_edge(jnp.concatenate(
        [edge_index_t_s[0], edge_index_i_s[0] + SNP]), spad, 0)
    sdst = _pad_edge(jnp.concatenate(
        [edge_index_t_s[1], edge_index_i_s[1] + SNP]), spad, nm)
    zeros_sn = jnp.zeros((SN2, EH), jnp.float32)
    p = _edge_prop(y0s, ssrc, sdst, zeros_sn, SN2, s_s)
    cs1, ys1 = _row_comb(_comb_b_body, 2, p[:SN2], p[SN2:], y0s, dinv_s)
    p = _edge_prop(ys1, ssrc, sdst, zeros_sn, SN2, s_s)
    cs2, _ = _row_comb(_comb_b_body, 2, p[:SN2], p[SN2:], ys1, dinv_s)
    mean_p = _tc_comb_e(cs1, cs2)

    s_m = _steps(nm)
    mpad = NW * G * s_m
    pm = _edge_prop(mean_p, _pad_edge(jnp.arange(nm, dtype=jnp.int32),
                                      mpad, 0),
                    _pad_edge(movie_indices, mpad, n), zeros_np, NP, s_m)

    ent = _mlp_proj([ent_q, pm[:NP], pm[NP:]], ep1_w1, ep1_b1, ep1_w2,
                    ep1_b2, ep2_w, ep2_b, grid=4)

    b, lt, tok = token_embeds.shape
    t2 = _mlp_proj([token_embeds.reshape(b * lt, tok)], tp1_w1, tp1_b1,
                   tp1_w2, tp1_b2, tp2_w, tp2_b)
    t2 = t2.reshape(b, lt, HIDDEN)
    eg = _sc_gather(ent, entity_ids.reshape(-1), b * 32, HIDDEN)
    e = eg.reshape(b, 32, HIDDEN)
    return _attention(t2, e, cross_w)


# speedup vs baseline: 5.6680x; 1.1305x over previous
import functools

import jax
import jax.numpy as jnp
from jax import lax
from jax.experimental import pallas as pl
from jax.experimental.pallas import tpu as pltpu
from jax.experimental.pallas import tpu_sc as plsc

HIDDEN = 256
EH = 128
N_REL = 8

NC, NSUB, G = 2, 16, 128
NW = NC * NSUB

NP = 10112
SNP = 5056
SN2 = 2 * SNP
BINS = 100352

_MESH = plsc.VectorSubcoreMesh(core_axis_name="c", subcore_axis_name="s")


def _sc_hist(idx3, ones_g, zeros_b, steps):
    bpt = BINS // NSUB

    def body(idx_h, ones_h, zero_h, out_h, idx_v, ones_v, acc_s, asem):
        cid = lax.axis_index("c")
        sid = lax.axis_index("s")
        wid = cid * NSUB + sid
        pltpu.sync_copy(idx_h.at[pl.ds(wid * steps, steps)], idx_v)
        pltpu.sync_copy(ones_h, ones_v)
        b0 = sid * bpt
        pltpu.sync_copy(zero_h.at[pl.ds(b0, bpt)], acc_s.at[pl.ds(b0, bpt)])
        plsc.subcore_barrier()

        def step(q, carry):
            for i in range(8):
                pltpu.async_copy(ones_v, acc_s.at[idx_v.at[q * 8 + i]],
                                 asem, add=True)
            for i in range(8):
                pltpu.make_async_copy(ones_v, acc_s.at[idx_v.at[q * 8 + i]],
                                      asem).wait()
            return carry

        lax.fori_loop(0, steps // 8, step, 0)
        plsc.subcore_barrier()
        pltpu.sync_copy(acc_s.at[pl.ds(b0, bpt)],
                        out_h.at[pl.ds(cid * BINS + b0, bpt)])

    k = pl.kernel(
        body,
        out_type=jax.ShapeDtypeStruct((NC * BINS,), jnp.float32),
        mesh=_MESH,
        scratch_types=[
            pltpu.VMEM((steps, G), jnp.int32),
            pltpu.VMEM((G,), jnp.float32),
            pltpu.VMEM_SHARED((BINS,), jnp.float32),
            pltpu.SemaphoreType.DMA,
        ],
    )
    return k(idx3, ones_g, zeros_b)


def _edge_prop(table, src3, dst3, zeros, n_pad, steps, inv=None, seg3=None):
    weighted = inv is not None
    rpt = n_pad // NSUB
    nh = src3.shape[0]
    s2 = src3.shape[2]

    def body(table_h, src_h, dst_h, zero_h, *rest):
        if weighted:
            (inv_h, seg_h, out_h, src_v, dst_v, buf0, buf1, acc_s, gs0, gs1,
             seg_v, w0, w1, ws0, ws1) = rest
        else:
            out_h, src_v, dst_v, buf0, buf1, acc_s, gs0, gs1 = rest
            w0 = w1 = ws0 = ws1 = None
        bufs = ((buf0, gs0, w0, ws0), (buf1, gs1, w1, ws1))
        cid = lax.axis_index("c")
        sid = lax.axis_index("s")
        wid = cid * NSUB + sid
        r0 = sid * rpt
        pltpu.sync_copy(zero_h.at[pl.ds(r0, rpt)], acc_s.at[pl.ds(r0, rpt)])
        plsc.subcore_barrier()

        def fire(j, buf, gsem, wv, wsem):
            pltpu.async_copy(table_h.at[src_v.at[j]], buf, gsem)
            if weighted:
                pltpu.async_copy(inv_h.at[seg_v.at[j]], wv, wsem)

        for h in range(nh):
            pltpu.sync_copy(src_h.at[h, wid], src_v)
            pltpu.sync_copy(dst_h.at[h, wid], dst_v)
            if weighted:
                pltpu.sync_copy(seg_h.at[h, wid], seg_v)
            fire(0, buf0, gs0, w0, ws0)

            def pair(p, carry):
                for k in (0, 1):
                    j = 2 * p + k
                    buf, gsem, wv, wsem = bufs[k]
                    nbuf, ngsem, nwv, nwsem = bufs[1 - k]

                    @pl.when(j + 1 < s2)
                    def _():
                        fire(j + 1, nbuf, ngsem, nwv, nwsem)

                    pltpu.make_async_copy(table_h.at[src_v.at[j]], buf,
                                          gsem).wait()
                    if weighted:
                        pltpu.make_async_copy(inv_h.at[seg_v.at[j]], wv,
                                              wsem).wait()

                        def srow(r, c2):
                            w16 = wv[pl.ds(r, 1)][0]
                            for q in range(EH // 16):
                                sl = pl.ds(q * 16, 16)
                                buf[r, sl] = buf[r, sl] * w16
                            return c2

                        lax.fori_loop(0, G, srow, 0, unroll=4)
                    pltpu.sync_copy(buf, acc_s.at[dst_v.at[j]], add=True)
                return carry

            lax.fori_loop(0, s2 // 2, pair, 0)
        plsc.subcore_barrier()
        pltpu.sync_copy(acc_s.at[pl.ds(r0, rpt)],
                        out_h.at[pl.ds(cid * n_pad + r0, rpt)])

    scratch = [
        pltpu.VMEM((s2, G), jnp.int32),
        pltpu.VMEM((s2, G), jnp.int32),
        pltpu.VMEM((G, EH), jnp.float32),
        pltpu.VMEM((G, EH), jnp.float32),
        pltpu.VMEM_SHARED((n_pad, EH), jnp.float32),
        pltpu.SemaphoreType.DMA,
        pltpu.SemaphoreType.DMA,
    ]
    if weighted:
        scratch += [
            pltpu.VMEM((s2, G), jnp.int32),
            pltpu.VMEM((G,), jnp.float32),
            pltpu.VMEM((G,), jnp.float32),
            pltpu.SemaphoreType.DMA,
            pltpu.SemaphoreType.DMA,
        ]
    k = pl.kernel(
        body,
        out_type=jax.ShapeDtypeStruct((NC * n_pad, EH), jnp.float32),
        mesh=_MESH,
        scratch_types=scratch,
    )
    args = (table, src3, dst3, zeros) + ((inv, seg3) if weighted else ())
    return k(*args)


def _sc_gather(table, idx, r_pad, d):
    rpw = r_pad // NW

    def body(table_h, idx_h, out_h, idx_v, rows_v, sem):
        cid = lax.axis_index("c")
        sid = lax.axis_index("s")
        wid = cid * NSUB + sid
        base = wid * rpw
        pltpu.sync_copy(idx_h.at[pl.ds(base, rpw)], idx_v)
        pltpu.async_copy(table_h.at[idx_v], rows_v, sem).wait()
        pltpu.sync_copy(rows_v, out_h.at[pl.ds(base, rpw)])

    k = pl.kernel(
        body,
        out_type=jax.ShapeDtypeStruct((r_pad, d), jnp.float32),
        mesh=_MESH,
        scratch_types=[
            pltpu.VMEM((rpw,), jnp.int32),
            pltpu.VMEM((rpw, d), jnp.float32),
            pltpu.SemaphoreType.DMA,
        ],
    )
    return k(table, idx)


def _rgcn_dense_body(comp_ref, basis_ref, x_ref, rootw_ref, rootb_ref,
                     xr_ref, rt_ref):
    r = pl.program_id(0)
    nr = comp_ref.shape[0]
    sel = (jax.lax.broadcasted_iota(jnp.int32, (1, nr), 1) == r)
    row = jnp.where(sel, 1.0, 0.0) @ comp_ref[...]
    W = jnp.tensordot(row, basis_ref[...], axes=((1,), (0,)))[0]
    x = x_ref[...]
    xr_ref[...] = (x @ W)[None]

    @pl.when(r == 0)
    def _():
        rt_ref[...] = x @ rootw_ref[...] + rootb_ref[...][None] + x


def _rgcn_dense(x, basis, comp, root_w, root_b):
    n = x.shape[0]
    return pl.pallas_call(
        _rgcn_dense_body,
        grid=(N_REL,),
        in_specs=[
            pl.BlockSpec(comp.shape, lambda r: (0, 0)),
            pl.BlockSpec(basis.shape, lambda r: (0, 0, 0)),
            pl.BlockSpec(x.shape, lambda r: (0, 0)),
            pl.BlockSpec(root_w.shape, lambda r: (0, 0)),
            pl.BlockSpec(root_b.shape, lambda r: (0,)),
        ],
        out_specs=[
            pl.BlockSpec((1, n, EH), lambda r: (r, 0, 0)),
            pl.BlockSpec((n, EH), lambda r: (0, 0)),
        ],
        out_shape=[
            jax.ShapeDtypeStruct((N_REL, n, EH), jnp.float32),
            jax.ShapeDtypeStruct((n, EH), jnp.float32),
        ],
    )(comp, basis, x, root_w, root_b)


def _norms_body(h_ref, out_ref):
    v = h_ref[0] + h_ref[1]
    r, c = v.shape
    pos = (jax.lax.broadcasted_iota(jnp.int32, (r, c), 0) * c
           + jax.lax.broadcasted_iota(jnp.int32, (r, c), 1))
    inv = 1.0 / jnp.maximum(v, 1.0)
    dinv = jax.lax.rsqrt(v + 1.0)
    out_ref[...] = jnp.where(pos < 80000, inv, dinv)


def _tc_norms(hist2):
    return pl.pallas_call(
        _norms_body,
        out_shape=jax.ShapeDtypeStruct(hist2.shape[1:], jnp.float32),
    )(hist2)


def _comb_a_body(p0_ref, p1_ref, rt_ref, dinv_ref, ent_ref, y_ref):
    ent = p0_ref[...] + p1_ref[...] + rt_ref[...]
    ent_ref[...] = ent
    y_ref[...] = dinv_ref[...] * ent


def _comb_b_body(p0_ref, p1_ref, y_ref, dinv_ref, c_ref, y2_ref):
    d = dinv_ref[...]
    c = d * (p0_ref[...] + p1_ref[...] + y_ref[...])
    c_ref[...] = c
    y2_ref[...] = d * c


def _row_comb(body, outs, *arrs):
    n = arrs[0].shape[0]
    blk = n // 4
    specs = [pl.BlockSpec((blk, a.shape[1]), lambda i: (i, 0)) for a in arrs]
    return pl.pallas_call(
        body,
        grid=(4,),
        in_specs=specs,
        out_specs=[pl.BlockSpec((blk, EH), lambda i: (i, 0))] * outs,
        out_shape=[jax.ShapeDtypeStruct((n, EH), jnp.float32)] * outs,
    )(*arrs)


def _comb_c_body(nf_ref, dinv_ref, out_ref):
    out_ref[...] = dinv_ref[...] * nf_ref[...]


def _tc_comb_c(nf, dinv_s):
    return pl.pallas_call(
        _comb_c_body,
        grid=(2,),
        in_specs=[
            pl.BlockSpec((SNP, EH), lambda i: (0, 0)),
            pl.BlockSpec((SNP, 1), lambda i: (i, 0)),
        ],
        out_specs=pl.BlockSpec((SNP, EH), lambda i: (i, 0)),
        out_shape=jax.ShapeDtypeStruct((SN2, EH), jnp.float32),
    )(nf, dinv_s)


def _comb_d_body(c1_ref, c2_ref, c3_ref, e0_ref, out_ref):
    out_ref[...] = 0.25 * (c1_ref[...] + c2_ref[...] + c3_ref[...]
                           + e0_ref[...])


def _comb_e_body(a_ref, b_ref, out_ref):
    out_ref[...] = 0.25 * (a_ref[0:SNP] + a_ref[SNP:SN2]
                           + b_ref[0:SNP] + b_ref[SNP:SN2])


def _tc_comb_e(cs1, cs2):
    return pl.pallas_call(
        _comb_e_body,
        out_shape=jax.ShapeDtypeStruct((SNP, EH), jnp.float32),
    )(cs1, cs2)


def _mlp_body(nx, x_refs_and_w):
    def body(*refs):
        xs = refs[:nx]
        w1_ref, b1_ref, w2_ref, b2_ref, w3_ref, b3_ref, out_ref = refs[nx:]
        x = xs[0][...]
        for xr in xs[1:]:
            x = x + xr[...]
        h = jnp.maximum(x @ w1_ref[...] + b1_ref[...][None], 0.0)
        h = h @ w2_ref[...] + b2_ref[...][None] + x
        out_ref[...] = h @ w3_ref[...] + b3_ref[...][None]
    return body


def _mlp_proj(xs, w1, b1, w2, b2, w3, b3, grid=1):
    n, din = xs[0].shape
    dout = w3.shape[1]
    blk = n // grid
    specs = [pl.BlockSpec((blk, din), lambda i: (i, 0)) for _ in xs]
    specs += [pl.BlockSpec(w.shape, lambda i, r=len(w.shape): (0,) * r)
              for w in (w1, b1, w2, b2, w3, b3)]
    return pl.pallas_call(
        _mlp_body(len(xs), None),
        grid=(grid,),
        in_specs=specs,
        out_specs=pl.BlockSpec((blk, dout), lambda i: (i, 0)),
        out_shape=jax.ShapeDtypeStruct((n, dout), jnp.float32),
    )(*xs, w1, b1, w2, b2, w3, b3)


def _attn_body(t_ref, e_ref, cw_ref, out_ref):
    t = t_ref[0]
    e = e_ref[0]
    a = (t @ cw_ref[...]) @ e.T * (1.0 / HIDDEN)
    a = a - jnp.max(a, axis=1, keepdims=True)
    ex = jnp.exp(a)
    ew = ex / jnp.sum(ex, axis=1, keepdims=True)
    out_ref[...] = (ew @ e + t)[None]


def _attention(t, e, cross_w):
    b, lt, _ = t.shape
    le = e.shape[1]
    return pl.pallas_call(
        _attn_body,
        grid=(b,),
        in_specs=[
            pl.BlockSpec((1, lt, HIDDEN), lambda i: (i, 0, 0)),
            pl.BlockSpec((1, le, HIDDEN), lambda i: (i, 0, 0)),
            pl.BlockSpec(cross_w.shape, lambda i: (0, 0)),
        ],
        out_specs=pl.BlockSpec((1, lt, HIDDEN), lambda i: (i, 0, 0)),
        out_shape=jax.ShapeDtypeStruct((b, lt, HIDDEN), jnp.float32),
    )(t, e, cross_w)


def _pad_edge(v, total, fill, halves=2):
    p = jnp.concatenate(
        [v, jnp.full((total - v.shape[0],), fill, jnp.int32)])
    s = total // (NW * G)
    if halves == 1:
        return p.reshape(-1, G)
    return p.reshape(NW, halves, s // halves, G).transpose(1, 0, 2, 3)


def _steps(e):
    return -(-e // (NW * G * 8)) * 8


def kernel(node_embeds, basis, comp, root_w, root_b, ep1_w1, ep1_b1, ep1_w2,
           ep1_b2, ep2_w, ep2_b, tp1_w1, tp1_b1, tp1_w2, tp1_b2, tp2_w, tp2_b,
           cross_w, token_embeds, entity_ids, edge_index, edge_type,
           edge_index_c, edge_index_t_s, edge_index_i_s, movie_indices):
    n = node_embeds.shape[0]
    nm = movie_indices.shape[0]
    x = jnp.pad(node_embeds, ((0, NP - n), (0, 0)))

    xr, rt = _rgcn_dense(x, basis, comp, root_w, root_b)
    xr_flat = xr.reshape(N_REL * NP, EH)

    src, dst, et = edge_index[0], edge_index[1], edge_type
    e_kg = src.shape[0]
    fe = et * NP + src
    seg = dst * N_REL + et

    hseg = jnp.concatenate([
        seg,
        80000 + edge_index_c[1],
        90000 + edge_index_t_s[1],
        95000 + edge_index_i_s[1],
    ])
    sh = _steps(hseg.shape[0])
    hidx = _pad_edge(hseg, NW * G * sh, BINS - 1, halves=1)
    ones_g = jnp.ones((G,), jnp.float32)
    zeros_b = jnp.zeros((BINS,), jnp.float32)
    hist2 = _sc_hist(hidx, ones_g, zeros_b, sh)
    norms = _tc_norms(hist2.reshape(2, BINS // 128, 128)).reshape(-1)
    inv = norms[:80000]
    dinv_c = jnp.pad(norms[80000:90000], (0, NP - n),
                     constant_values=1.0)[:, None]
    dinv_s = jnp.concatenate([
        jnp.pad(norms[90000:95000], (0, SNP - nm), constant_values=1.0),
        jnp.pad(norms[95000:100000], (0, SNP - nm), constant_values=1.0),
    ])[:, None]

    s_kg = _steps(e_kg)
    epad = NW * G * s_kg
    zeros_np = jnp.zeros((NP, EH), jnp.float32)
    pr = _edge_prop(xr_flat, _pad_edge(fe, epad, 0), _pad_edge(dst, epad, n),
                    zeros_np, NP, s_kg, inv=inv,
                    seg3=_pad_edge(seg, epad, 0))
    ent0, y0c = _row_comb(_comb_a_body, 2, pr[:NP], pr[NP:], rt, dinv_c)

    e_c = edge_index_c.shape[1]
    s_c = _steps(e_c)
    cpad = NW * G * s_c
    csrc = _pad_edge(edge_index_c[0], cpad, 0)
    cdst = _pad_edge(edge_index_c[1], cpad, n)
    p = _edge_prop(y0c, csrc, cdst, zeros_np, NP, s_c)
    c1, y1 = _row_comb(_comb_b_body, 2, p[:NP], p[NP:], y0c, dinv_c)
    p = _edge_prop(y1, csrc, cdst, zeros_np, NP, s_c)
    c2, y2 = _row_comb(_comb_b_body, 2, p[:NP], p[NP:], y1, dinv_c)
    p = _edge_prop(y2, csrc, cdst, zeros_np, NP, s_c)
    c3, _ = _row_comb(_comb_b_body, 2, p[:NP], p[NP:], y2, dinv_c)
    ent_q = _row_comb(_comb_d_body, 1, c1, c2, c3, ent0)[0]

    mi_pad = jnp.concatenate(
        [movie_indices, jnp.zeros((5120 - nm,), jnp.int32)])
    nf = _sc_gather(ent0, mi_pad, 5120, EH)[:SNP]
    y0s = _tc_comb_c(nf, dinv_s)
    e_s = edge_index_t_s.shape[1]
    s_s = _steps(2 * e_s)
    spad = NW * G * s_s
    ssrc = _pad_edge(jnp.concatenate(
        [edge_index_t_s[0], edge_index_i_s[0] + SNP]), spad, 0)
    sdst = _pad_edge(jnp.concatenate(
        [edge_index_t_s[1], edge_index_i_s[1] + SNP]), spad, nm)
    zeros_sn = jnp.zeros((SN2, EH), jnp.float32)
    p = _edge_prop(y0s, ssrc, sdst, zeros_sn, SN2, s_s)
    cs1, ys1 = _row_comb(_comb_b_body, 2, p[:SN2], p[SN2:], y0s, dinv_s)
    p = _edge_prop(ys1, ssrc, sdst, zeros_sn, SN2, s_s)
    cs2, _ = _row_comb(_comb_b_body, 2, p[:SN2], p[SN2:], ys1, dinv_s)
    mean_p = _tc_comb_e(cs1, cs2)

    s_m = _steps(nm)
    mpad = NW * G * s_m
    pm = _edge_prop(mean_p, _pad_edge(jnp.arange(nm, dtype=jnp.int32),
                                      mpad, 0),
                    _pad_edge(movie_indices, mpad, n), zeros_np, NP, s_m)

    ent = _mlp_proj([ent_q, pm[:NP], pm[NP:]], ep1_w1, ep1_b1, ep1_w2,
                    ep1_b2, ep2_w, ep2_b, grid=4)

    b, lt, tok = token_embeds.shape
    t2 = _mlp_proj([token_embeds.reshape(b * lt, tok)], tp1_w1, tp1_b1,
                   tp1_w2, tp1_b2, tp2_w, tp2_b)
    t2 = t2.reshape(b, lt, HIDDEN)
    eg = _sc_gather(ent, entity_ids.reshape(-1), b * 32, HIDDEN)
    e = eg.reshape(b, 32, HIDDEN)
    return _attention(t2, e, cross_w)
